# baseline (speedup 1.0000x reference)
import numpy as np

import jax
import jax.numpy as jnp
from jax import lax
from jax.experimental import pallas as pl
from jax.experimental.pallas import tpu as pltpu
from jax.experimental.pallas import tpu_sc as plsc

N = 10000
NP = 10240
E = 320000
EP = 327680
NWORK = 32
CHUNKS = 80
CW = 128
RPT = NP // 16
DH = 16
DI = 128
DO = 128
RB = 2048

_PAD_EDGES = np.stack([
    (N + (np.arange(EP - E) * 7 + 3) % (NP - N)).astype(np.int32),
    (N + np.arange(EP - E) % (NP - N)).astype(np.int32),
])


def _mesh():
    return plsc.VectorSubcoreMesh(
        core_axis_name="c", subcore_axis_name="s", num_cores=2, num_subcores=16
    )


_SC_PARAMS = pltpu.CompilerParams(use_tc_tiling_on_sc=False)


def _rsqrt16(x):
    xi = lax.bitcast_convert_type(x, jnp.int32)
    yi = jnp.int32(0x5F3759DF) - (xi >> 1)
    y = lax.bitcast_convert_type(yi, jnp.float32)
    for _ in range(3):
        y = y * (1.5 - 0.5 * x * y * y)
    return y


def _edge_pipeline(srcv, dstv, bufs, gsh, accum, sg, ss):
    nb = len(bufs)
    ah = nb // 2
    for k in range(ah):
        pltpu.async_copy(gsh.at[srcv.at[k]], bufs[k], sg[k])

    @pl.loop(0, CHUNKS, step=nb)
    def _pipe(j0):
        for b in range(nb):
            j = j0 + b
            bn = (b + ah) % nb

            @pl.when(jnp.logical_and(j + ah < CHUNKS, j >= ah))
            def _():
                pltpu.make_async_copy(bufs[bn], accum.at[dstv.at[0]], ss[bn]).wait()

            @pl.when(j + ah < CHUNKS)
            def _():
                pltpu.async_copy(gsh.at[srcv.at[j + ah]], bufs[bn], sg[bn])

            pltpu.make_async_copy(gsh.at[srcv.at[0]], bufs[b], sg[b]).wait()
            pltpu.async_copy(bufs[b], accum.at[dstv.at[j]], ss[b], add=True)

    for b in range(nb):
        pltpu.make_async_copy(bufs[b], accum.at[dstv.at[0]], ss[b]).wait()


_NB = 8
_ROW_BUFS = [pltpu.VMEM((CW, DH), jnp.float32)] * _NB
_PIPE_SEMS = [pltpu.SemaphoreType.DMA] * (2 * _NB)


def _deg_body(er_hbm, ones_hbm, zeros_hbm, out_hbm, dstv, ones_rows, accum, sem, sem2):
    cid = lax.axis_index("c")
    sid = lax.axis_index("s")
    wid = cid * 16 + sid
    sl = pl.ds(sid * RPT, RPT)

    pltpu.async_copy(ones_hbm, ones_rows, sem2)
    pltpu.sync_copy(zeros_hbm, accum.at[sl])
    pltpu.sync_copy(er_hbm.at[1, wid], dstv)
    pltpu.make_async_copy(ones_hbm, ones_rows, sem2).wait()
    plsc.subcore_barrier()

    def ch(j, _):
        pltpu.async_copy(ones_rows, accum.at[dstv.at[j]], sem, add=True)
        return 0

    lax.fori_loop(0, CHUNKS, ch, 0)

    def dr(j, _):
        pltpu.make_async_copy(ones_rows, accum.at[dstv.at[0]], sem).wait()
        return 0

    lax.fori_loop(0, CHUNKS, dr, 0)
    plsc.subcore_barrier()

    pltpu.sync_copy(accum.at[sl], out_hbm.at[cid, sl])


_sc_deg = pl.kernel(
    _deg_body,
    out_type=jax.ShapeDtypeStruct((2, NP, DH), jnp.float32),
    mesh=_mesh(),
    scratch_types=[
        pltpu.VMEM((CHUNKS, CW), jnp.int32),
        pltpu.VMEM((CW, DH), jnp.float32),
        pltpu.VMEM_SHARED((NP, DH), jnp.float32),
        pltpu.SemaphoreType.DMA,
        pltpu.SemaphoreType.DMA,
    ],
    compiler_params=_SC_PARAMS,
)


def _agg1_body(
    h_hbm, d_hbm, er_hbm, zeros_hbm,
    s1_hbm, g1_hbm, dv_hbm,
    srcv, dstv, vh, vd0, vd1, gsh, accum,
    *bufs_sems,
):
    bufs = bufs_sems[:_NB]
    sg = bufs_sems[_NB:2 * _NB]
    ss = bufs_sems[2 * _NB:]
    g0, g1s, g2s = sg[0], sg[1], sg[2]
    s3 = ss[_NB - 1]
    cid = lax.axis_index("c")
    sid = lax.axis_index("s")
    wid = cid * 16 + sid
    sl = pl.ds(sid * RPT, RPT)

    pltpu.async_copy(h_hbm.at[sl], vh, g0)
    pltpu.async_copy(d_hbm.at[0, sl], vd0, g1s)
    pltpu.async_copy(d_hbm.at[1, sl], vd1, g2s)
    pltpu.sync_copy(zeros_hbm, accum.at[sl])
    pltpu.sync_copy(er_hbm.at[0, wid], srcv)
    pltpu.sync_copy(er_hbm.at[1, wid], dstv)
    pltpu.make_async_copy(h_hbm.at[sl], vh, g0).wait()
    pltpu.make_async_copy(d_hbm.at[0, sl], vd0, g1s).wait()
    pltpu.make_async_copy(d_hbm.at[1, sl], vd1, g2s).wait()

    def cb(i, _):
        deg = 1.0 + vd0[i, :] + vd1[i, :]
        dv = _rsqrt16(deg)
        vh[i, :] = vh[i, :] * dv
        vd1[i, :] = dv
        return 0

    lax.fori_loop(0, RPT, cb, 0, unroll=8)

    pltpu.async_copy(vh, g1_hbm.at[sl], s3)
    pltpu.async_copy(vd1, dv_hbm.at[sl], s3)
    pltpu.sync_copy(vh, gsh.at[sl])
    plsc.subcore_barrier()
    pltpu.make_async_copy(vh, g1_hbm.at[sl], s3).wait()
    pltpu.make_async_copy(vd1, dv_hbm.at[sl], s3).wait()
    _edge_pipeline(srcv, dstv, bufs, gsh, accum, sg, ss)
    plsc.subcore_barrier()
    pltpu.sync_copy(accum.at[sl], s1_hbm.at[cid, sl])


_sc_agg1 = pl.kernel(
    _agg1_body,
    out_type=[
        jax.ShapeDtypeStruct((2, NP, DH), jnp.float32),
        jax.ShapeDtypeStruct((NP, DH), jnp.float32),
        jax.ShapeDtypeStruct((NP, DH), jnp.float32),
    ],
    mesh=_mesh(),
    scratch_types=[
        pltpu.VMEM((CHUNKS, CW), jnp.int32),
        pltpu.VMEM((CHUNKS, CW), jnp.int32),
        pltpu.VMEM((RPT, DH), jnp.float32),
        pltpu.VMEM((RPT, DH), jnp.float32),
        pltpu.VMEM((RPT, DH), jnp.float32),
        pltpu.VMEM_SHARED((NP, DH), jnp.float32),
        pltpu.VMEM_SHARED((NP, DH), jnp.float32),
        *_ROW_BUFS,
        *_PIPE_SEMS,
    ],
    compiler_params=_SC_PARAMS,
)


def _agg2_body(
    s1_hbm, g1_hbm, dv_hbm, b1_hbm, er_hbm, zeros_hbm,
    s2_hbm, g2_hbm,
    srcv, dstv, vg, vs0, vs1, vdv, b1v, gsh, accum,
    *bufs_sems,
):
    bufs = bufs_sems[:_NB]
    sg = bufs_sems[_NB:2 * _NB]
    ss = bufs_sems[2 * _NB:]
    g0, g1s, g2s, g3 = sg[0], sg[1], sg[2], sg[3]
    s3 = ss[_NB - 1]
    cid = lax.axis_index("c")
    sid = lax.axis_index("s")
    wid = cid * 16 + sid
    sl = pl.ds(sid * RPT, RPT)

    pltpu.async_copy(g1_hbm.at[sl], vg, g0)
    pltpu.async_copy(s1_hbm.at[0, sl], vs0, g1s)
    pltpu.async_copy(s1_hbm.at[1, sl], vs1, g2s)
    pltpu.async_copy(dv_hbm.at[sl], vdv, g3)
    pltpu.sync_copy(zeros_hbm, accum.at[sl])
    pltpu.sync_copy(er_hbm.at[0, wid], srcv)
    pltpu.sync_copy(er_hbm.at[1, wid], dstv)
    pltpu.sync_copy(b1_hbm, b1v)
    b1 = b1v[...]
    pltpu.make_async_copy(g1_hbm.at[sl], vg, g0).wait()
    pltpu.make_async_copy(s1_hbm.at[0, sl], vs0, g1s).wait()
    pltpu.make_async_copy(s1_hbm.at[1, sl], vs1, g2s).wait()
    pltpu.make_async_copy(dv_hbm.at[sl], vdv, g3).wait()

    def cb(i, _):
        dv = vdv[i, :]
        o1 = dv * (vs0[i, :] + vs1[i, :] + vg[i, :]) + b1
        vg[i, :] = jnp.maximum(o1, 0.0) * dv
        return 0

    lax.fori_loop(0, RPT, cb, 0, unroll=8)

    pltpu.async_copy(vg, g2_hbm.at[sl], s3)
    pltpu.sync_copy(vg, gsh.at[sl])
    plsc.subcore_barrier()
    pltpu.make_async_copy(vg, g2_hbm.at[sl], s3).wait()
    _edge_pipeline(srcv, dstv, bufs, gsh, accum, sg, ss)
    plsc.subcore_barrier()
    pltpu.sync_copy(accum.at[sl], s2_hbm.at[cid, sl])


_sc_agg2 = pl.kernel(
    _agg2_body,
    out_type=[
        jax.ShapeDtypeStruct((2, NP, DH), jnp.float32),
        jax.ShapeDtypeStruct((NP, DH), jnp.float32),
    ],
    mesh=_mesh(),
    scratch_types=[
        pltpu.VMEM((CHUNKS, CW), jnp.int32),
        pltpu.VMEM((CHUNKS, CW), jnp.int32),
        pltpu.VMEM((RPT, DH), jnp.float32),
        pltpu.VMEM((RPT, DH), jnp.float32),
        pltpu.VMEM((RPT, DH), jnp.float32),
        pltpu.VMEM((RPT, DH), jnp.float32),
        pltpu.VMEM((DH,), jnp.float32),
        pltpu.VMEM_SHARED((NP, DH), jnp.float32),
        pltpu.VMEM_SHARED((NP, DH), jnp.float32),
        *_ROW_BUFS,
        *_PIPE_SEMS,
    ],
    compiler_params=_SC_PARAMS,
)


def _fin_body(s2_hbm, g2_hbm, dv_hbm, aggf_hbm, vs0, vs1, vg, vdv, g0, g1s, g2s, g3):
    sid = lax.axis_index("s")
    cid = lax.axis_index("c")
    w = sid * 2 + cid
    sl = pl.ds(w * (RPT // 2), RPT // 2)

    pltpu.async_copy(s2_hbm.at[0, sl], vs0, g0)
    pltpu.async_copy(s2_hbm.at[1, sl], vs1, g1s)
    pltpu.async_copy(g2_hbm.at[sl], vg, g2s)
    pltpu.async_copy(dv_hbm.at[sl], vdv, g3)
    pltpu.make_async_copy(s2_hbm.at[0, sl], vs0, g0).wait()
    pltpu.make_async_copy(s2_hbm.at[1, sl], vs1, g1s).wait()
    pltpu.make_async_copy(g2_hbm.at[sl], vg, g2s).wait()
    pltpu.make_async_copy(dv_hbm.at[sl], vdv, g3).wait()

    def cb(i, _):
        vg[i, :] = vdv[i, :] * (vs0[i, :] + vs1[i, :] + vg[i, :])
        return 0

    lax.fori_loop(0, RPT // 2, cb, 0, unroll=8)
    pltpu.sync_copy(vg, aggf_hbm.at[sl])


_sc_fin = pl.kernel(
    _fin_body,
    out_type=jax.ShapeDtypeStruct((NP, DH), jnp.float32),
    mesh=_mesh(),
    scratch_types=[
        pltpu.VMEM((RPT // 2, DH), jnp.float32),
        pltpu.VMEM((RPT // 2, DH), jnp.float32),
        pltpu.VMEM((RPT // 2, DH), jnp.float32),
        pltpu.VMEM((RPT // 2, DH), jnp.float32),
        pltpu.SemaphoreType.DMA,
        pltpu.SemaphoreType.DMA,
        pltpu.SemaphoreType.DMA,
        pltpu.SemaphoreType.DMA,
    ],
    compiler_params=_SC_PARAMS,
)


def _tc_a_body(x_ref, w_ref, h_ref):
    h_ref[...] = jnp.dot(x_ref[...], w_ref[...], preferred_element_type=jnp.float32)


def _tc_a(x, W1):
    return pl.pallas_call(
        _tc_a_body,
        grid=(NP // RB,),
        in_specs=[
            pl.BlockSpec((RB, DI), lambda i: (i, 0)),
            pl.BlockSpec((DI, DH), lambda i: (0, 0)),
        ],
        out_specs=pl.BlockSpec((RB, DH), lambda i: (i, 0)),
        out_shape=jax.ShapeDtypeStruct((NP, DH), jnp.float32),
    )(x, W1)


def _tc_c_body(a_ref, w_ref, b_ref, o_ref):
    o_ref[...] = (
        jnp.dot(a_ref[...], w_ref[...], preferred_element_type=jnp.float32)
        + b_ref[...]
    )


def _tc_c(aggf, W2, b2):
    return pl.pallas_call(
        _tc_c_body,
        grid=(NP // RB,),
        in_specs=[
            pl.BlockSpec((RB, DH), lambda i: (i, 0)),
            pl.BlockSpec((DH, DO), lambda i: (0, 0)),
            pl.BlockSpec((1, DO), lambda i: (0, 0)),
        ],
        out_specs=pl.BlockSpec((RB, DO), lambda i: (i, 0)),
        out_shape=jax.ShapeDtypeStruct((N, DO), jnp.float32),
    )(aggf, W2, b2)


def kernel(x, edge_index, W1, b1, W2, b2):
    er = jnp.concatenate(
        [edge_index, jnp.asarray(_PAD_EDGES)], axis=1
    ).reshape(2, NWORK, CHUNKS, CW)
    zeros_c = jnp.zeros((RPT, DH), jnp.float32)
    ones_c = jnp.ones((CW, DH), jnp.float32)

    h = _tc_a(x, W1)
    degp = _sc_deg(er, ones_c, zeros_c)
    s1, g1, dv = _sc_agg1(h, degp, er, zeros_c)
    s2, g2 = _sc_agg2(s1, g1, dv, b1, er, zeros_c)
    aggf = _sc_fin(s2, g2, dv)
    return _tc_c(aggf, W2, b2.reshape(1, DO))


# speedup vs baseline: 1.0032x; 1.0032x over previous
import numpy as np

import jax
import jax.numpy as jnp
from jax import lax
from jax.experimental import pallas as pl
from jax.experimental.pallas import tpu as pltpu
from jax.experimental.pallas import tpu_sc as plsc

N = 10000
NP = 10240
E = 320000
EP = 327680
NWORK = 32
CHUNKS = 80
CW = 128
RPT = NP // 16
DH = 16
DI = 128
DO = 128
RB = 2048

_PAD_EDGES = np.stack([
    (N + (np.arange(EP - E) * 7 + 3) % (NP - N)).astype(np.int32),
    (N + np.arange(EP - E) % (NP - N)).astype(np.int32),
])


def _mesh():
    return plsc.VectorSubcoreMesh(
        core_axis_name="c", subcore_axis_name="s", num_cores=2, num_subcores=16
    )


_SC_PARAMS = pltpu.CompilerParams(use_tc_tiling_on_sc=False)


def _rsqrt16(x):
    xi = lax.bitcast_convert_type(x, jnp.int32)
    yi = jnp.int32(0x5F3759DF) - (xi >> 1)
    y = lax.bitcast_convert_type(yi, jnp.float32)
    for _ in range(3):
        y = y * (1.5 - 0.5 * x * y * y)
    return y


def _edge_pipeline(srcv, dstv, bufs, gsh, accum, sg, ss):
    nb = len(bufs)
    ah = nb // 2
    for k in range(ah):
        pltpu.async_copy(gsh.at[srcv.at[k]], bufs[k], sg[k])

    @pl.loop(0, CHUNKS, step=nb)
    def _pipe(j0):
        for b in range(nb):
            j = j0 + b
            bn = (b + ah) % nb

            @pl.when(jnp.logical_and(j + ah < CHUNKS, j >= ah))
            def _():
                pltpu.make_async_copy(bufs[bn], accum.at[dstv.at[0]], ss[bn]).wait()

            @pl.when(j + ah < CHUNKS)
            def _():
                pltpu.async_copy(gsh.at[srcv.at[j + ah]], bufs[bn], sg[bn])

            pltpu.make_async_copy(gsh.at[srcv.at[0]], bufs[b], sg[b]).wait()
            pltpu.async_copy(bufs[b], accum.at[dstv.at[j]], ss[b], add=True)

    for b in range(nb):
        pltpu.make_async_copy(bufs[b], accum.at[dstv.at[0]], ss[b]).wait()


_NB = 4
_ROW_BUFS = [pltpu.VMEM((CW, DH), jnp.float32)] * _NB
_PIPE_SEMS = [pltpu.SemaphoreType.DMA] * (2 * _NB)


def _deg_body(er_hbm, ones_hbm, zeros_hbm, out_hbm, dstv, ones_rows, accum, sem, sem2):
    cid = lax.axis_index("c")
    sid = lax.axis_index("s")
    wid = cid * 16 + sid
    sl = pl.ds(sid * RPT, RPT)

    pltpu.async_copy(ones_hbm, ones_rows, sem2)
    pltpu.sync_copy(zeros_hbm, accum.at[sl])
    pltpu.sync_copy(er_hbm.at[1, wid], dstv)
    pltpu.make_async_copy(ones_hbm, ones_rows, sem2).wait()
    plsc.subcore_barrier()

    def ch(j, _):
        pltpu.async_copy(ones_rows, accum.at[dstv.at[j]], sem, add=True)
        return 0

    lax.fori_loop(0, CHUNKS, ch, 0)

    def dr(j, _):
        pltpu.make_async_copy(ones_rows, accum.at[dstv.at[0]], sem).wait()
        return 0

    lax.fori_loop(0, CHUNKS, dr, 0)
    plsc.subcore_barrier()

    pltpu.sync_copy(accum.at[sl], out_hbm.at[cid, sl])


_sc_deg = pl.kernel(
    _deg_body,
    out_type=jax.ShapeDtypeStruct((2, NP, DH), jnp.float32),
    mesh=_mesh(),
    scratch_types=[
        pltpu.VMEM((CHUNKS, CW), jnp.int32),
        pltpu.VMEM((CW, DH), jnp.float32),
        pltpu.VMEM_SHARED((NP, DH), jnp.float32),
        pltpu.SemaphoreType.DMA,
        pltpu.SemaphoreType.DMA,
    ],
    compiler_params=_SC_PARAMS,
)


def _agg1_body(
    h_hbm, d_hbm, er_hbm, zeros_hbm,
    s1_hbm, g1_hbm, dv_hbm,
    srcv, dstv, vh, vd0, vd1, gsh, accum,
    *bufs_sems,
):
    bufs = bufs_sems[:_NB]
    sg = bufs_sems[_NB:2 * _NB]
    ss = bufs_sems[2 * _NB:]
    g0, g1s, g2s = sg[0], sg[1], sg[2]
    s3 = ss[_NB - 1]
    cid = lax.axis_index("c")
    sid = lax.axis_index("s")
    wid = cid * 16 + sid
    sl = pl.ds(sid * RPT, RPT)

    pltpu.async_copy(h_hbm.at[sl], vh, g0)
    pltpu.async_copy(d_hbm.at[0, sl], vd0, g1s)
    pltpu.async_copy(d_hbm.at[1, sl], vd1, g2s)
    pltpu.sync_copy(zeros_hbm, accum.at[sl])
    pltpu.sync_copy(er_hbm.at[0, wid], srcv)
    pltpu.sync_copy(er_hbm.at[1, wid], dstv)
    pltpu.make_async_copy(h_hbm.at[sl], vh, g0).wait()
    pltpu.make_async_copy(d_hbm.at[0, sl], vd0, g1s).wait()
    pltpu.make_async_copy(d_hbm.at[1, sl], vd1, g2s).wait()

    def cb(i, _):
        deg = 1.0 + vd0[i, :] + vd1[i, :]
        dv = _rsqrt16(deg)
        vh[i, :] = vh[i, :] * dv
        vd1[i, :] = dv
        return 0

    lax.fori_loop(0, RPT, cb, 0, unroll=8)

    pltpu.async_copy(vh, g1_hbm.at[sl], s3)
    pltpu.async_copy(vd1, dv_hbm.at[sl], s3)
    pltpu.sync_copy(vh, gsh.at[sl])
    plsc.subcore_barrier()
    pltpu.make_async_copy(vh, g1_hbm.at[sl], s3).wait()
    pltpu.make_async_copy(vd1, dv_hbm.at[sl], s3).wait()
    _edge_pipeline(srcv, dstv, bufs, gsh, accum, sg, ss)
    plsc.subcore_barrier()
    pltpu.sync_copy(accum.at[sl], s1_hbm.at[cid, sl])


_sc_agg1 = pl.kernel(
    _agg1_body,
    out_type=[
        jax.ShapeDtypeStruct((2, NP, DH), jnp.float32),
        jax.ShapeDtypeStruct((NP, DH), jnp.float32),
        jax.ShapeDtypeStruct((NP, DH), jnp.float32),
    ],
    mesh=_mesh(),
    scratch_types=[
        pltpu.VMEM((CHUNKS, CW), jnp.int32),
        pltpu.VMEM((CHUNKS, CW), jnp.int32),
        pltpu.VMEM((RPT, DH), jnp.float32),
        pltpu.VMEM((RPT, DH), jnp.float32),
        pltpu.VMEM((RPT, DH), jnp.float32),
        pltpu.VMEM_SHARED((NP, DH), jnp.float32),
        pltpu.VMEM_SHARED((NP, DH), jnp.float32),
        *_ROW_BUFS,
        *_PIPE_SEMS,
    ],
    compiler_params=_SC_PARAMS,
)


def _agg2_body(
    s1_hbm, g1_hbm, dv_hbm, b1_hbm, er_hbm, zeros_hbm,
    s2_hbm, g2_hbm,
    srcv, dstv, vg, vs0, vs1, vdv, b1v, gsh, accum,
    *bufs_sems,
):
    bufs = bufs_sems[:_NB]
    sg = bufs_sems[_NB:2 * _NB]
    ss = bufs_sems[2 * _NB:]
    g0, g1s, g2s, g3 = sg[0], sg[1], sg[2], sg[3]
    s3 = ss[_NB - 1]
    cid = lax.axis_index("c")
    sid = lax.axis_index("s")
    wid = cid * 16 + sid
    sl = pl.ds(sid * RPT, RPT)

    pltpu.async_copy(g1_hbm.at[sl], vg, g0)
    pltpu.async_copy(s1_hbm.at[0, sl], vs0, g1s)
    pltpu.async_copy(s1_hbm.at[1, sl], vs1, g2s)
    pltpu.async_copy(dv_hbm.at[sl], vdv, g3)
    pltpu.sync_copy(zeros_hbm, accum.at[sl])
    pltpu.sync_copy(er_hbm.at[0, wid], srcv)
    pltpu.sync_copy(er_hbm.at[1, wid], dstv)
    pltpu.sync_copy(b1_hbm, b1v)
    b1 = b1v[...]
    pltpu.make_async_copy(g1_hbm.at[sl], vg, g0).wait()
    pltpu.make_async_copy(s1_hbm.at[0, sl], vs0, g1s).wait()
    pltpu.make_async_copy(s1_hbm.at[1, sl], vs1, g2s).wait()
    pltpu.make_async_copy(dv_hbm.at[sl], vdv, g3).wait()

    def cb(i, _):
        dv = vdv[i, :]
        o1 = dv * (vs0[i, :] + vs1[i, :] + vg[i, :]) + b1
        vg[i, :] = jnp.maximum(o1, 0.0) * dv
        return 0

    lax.fori_loop(0, RPT, cb, 0, unroll=8)

    pltpu.async_copy(vg, g2_hbm.at[sl], s3)
    pltpu.sync_copy(vg, gsh.at[sl])
    plsc.subcore_barrier()
    pltpu.make_async_copy(vg, g2_hbm.at[sl], s3).wait()
    _edge_pipeline(srcv, dstv, bufs, gsh, accum, sg, ss)
    plsc.subcore_barrier()
    pltpu.sync_copy(accum.at[sl], s2_hbm.at[cid, sl])


_sc_agg2 = pl.kernel(
    _agg2_body,
    out_type=[
        jax.ShapeDtypeStruct((2, NP, DH), jnp.float32),
        jax.ShapeDtypeStruct((NP, DH), jnp.float32),
    ],
    mesh=_mesh(),
    scratch_types=[
        pltpu.VMEM((CHUNKS, CW), jnp.int32),
        pltpu.VMEM((CHUNKS, CW), jnp.int32),
        pltpu.VMEM((RPT, DH), jnp.float32),
        pltpu.VMEM((RPT, DH), jnp.float32),
        pltpu.VMEM((RPT, DH), jnp.float32),
        pltpu.VMEM((RPT, DH), jnp.float32),
        pltpu.VMEM((DH,), jnp.float32),
        pltpu.VMEM_SHARED((NP, DH), jnp.float32),
        pltpu.VMEM_SHARED((NP, DH), jnp.float32),
        *_ROW_BUFS,
        *_PIPE_SEMS,
    ],
    compiler_params=_SC_PARAMS,
)


def _fin_body(s2_hbm, g2_hbm, dv_hbm, aggf_hbm, vs0, vs1, vg, vdv, g0, g1s, g2s, g3):
    sid = lax.axis_index("s")
    cid = lax.axis_index("c")
    w = sid * 2 + cid
    sl = pl.ds(w * (RPT // 2), RPT // 2)

    pltpu.async_copy(s2_hbm.at[0, sl], vs0, g0)
    pltpu.async_copy(s2_hbm.at[1, sl], vs1, g1s)
    pltpu.async_copy(g2_hbm.at[sl], vg, g2s)
    pltpu.async_copy(dv_hbm.at[sl], vdv, g3)
    pltpu.make_async_copy(s2_hbm.at[0, sl], vs0, g0).wait()
    pltpu.make_async_copy(s2_hbm.at[1, sl], vs1, g1s).wait()
    pltpu.make_async_copy(g2_hbm.at[sl], vg, g2s).wait()
    pltpu.make_async_copy(dv_hbm.at[sl], vdv, g3).wait()

    def cb(i, _):
        vg[i, :] = vdv[i, :] * (vs0[i, :] + vs1[i, :] + vg[i, :])
        return 0

    lax.fori_loop(0, RPT // 2, cb, 0, unroll=8)
    pltpu.sync_copy(vg, aggf_hbm.at[sl])


_sc_fin = pl.kernel(
    _fin_body,
    out_type=jax.ShapeDtypeStruct((NP, DH), jnp.float32),
    mesh=_mesh(),
    scratch_types=[
        pltpu.VMEM((RPT // 2, DH), jnp.float32),
        pltpu.VMEM((RPT // 2, DH), jnp.float32),
        pltpu.VMEM((RPT // 2, DH), jnp.float32),
        pltpu.VMEM((RPT // 2, DH), jnp.float32),
        pltpu.SemaphoreType.DMA,
        pltpu.SemaphoreType.DMA,
        pltpu.SemaphoreType.DMA,
        pltpu.SemaphoreType.DMA,
    ],
    compiler_params=_SC_PARAMS,
)


def _tc_a_body(x_ref, w_ref, h_ref):
    h_ref[...] = jnp.dot(x_ref[...], w_ref[...], preferred_element_type=jnp.float32)


def _tc_a(x, W1):
    return pl.pallas_call(
        _tc_a_body,
        grid=(NP // RB,),
        in_specs=[
            pl.BlockSpec((RB, DI), lambda i: (i, 0)),
            pl.BlockSpec((DI, DH), lambda i: (0, 0)),
        ],
        out_specs=pl.BlockSpec((RB, DH), lambda i: (i, 0)),
        out_shape=jax.ShapeDtypeStruct((NP, DH), jnp.float32),
    )(x, W1)


def _tc_c_body(a_ref, w_ref, b_ref, o_ref):
    o_ref[...] = (
        jnp.dot(a_ref[...], w_ref[...], preferred_element_type=jnp.float32)
        + b_ref[...]
    )


def _tc_c(aggf, W2, b2):
    return pl.pallas_call(
        _tc_c_body,
        grid=(NP // RB,),
        in_specs=[
            pl.BlockSpec((RB, DH), lambda i: (i, 0)),
            pl.BlockSpec((DH, DO), lambda i: (0, 0)),
            pl.BlockSpec((1, DO), lambda i: (0, 0)),
        ],
        out_specs=pl.BlockSpec((RB, DO), lambda i: (i, 0)),
        out_shape=jax.ShapeDtypeStruct((N, DO), jnp.float32),
    )(aggf, W2, b2)


def kernel(x, edge_index, W1, b1, W2, b2):
    er = jnp.concatenate(
        [edge_index, jnp.asarray(_PAD_EDGES)], axis=1
    ).reshape(2, NWORK, CHUNKS, CW)
    zeros_c = jnp.zeros((RPT, DH), jnp.float32)
    ones_c = jnp.ones((CW, DH), jnp.float32)

    h = _tc_a(x, W1)
    degp = _sc_deg(er, ones_c, zeros_c)
    s1, g1, dv = _sc_agg1(h, degp, er, zeros_c)
    s2, g2 = _sc_agg2(s1, g1, dv, b1, er, zeros_c)
    aggf = _sc_fin(s2, g2, dv)
    return _tc_c(aggf, W2, b2.reshape(1, DO))
